# Initial kernel scaffold; baseline (speedup 1.0000x reference)
import jax
import jax.numpy as jnp
from jax.experimental import pallas as pl


def kernel(x, edge_index, hyperedge_index, pos_edges, neg_edges, W1, b1, W2, b2, Wh1, bh1, Wh2, bh2, gate, Wdec, Pw1, Pb1, Pw2, Pb2):
    raise NotImplementedError("write your pallas kernel here")


# speedup vs baseline: 1.4724x; 1.4724x over previous
import functools

import jax
import jax.numpy as jnp
from jax.experimental import pallas as pl


def _mm_body(a_ref, w_ref, b_ref, o_ref, *, act, scale_rows):
    acc = jnp.dot(a_ref[...], w_ref[...], preferred_element_type=jnp.float32)
    acc = acc + b_ref[...]
    if act == "elu":
        acc = jnp.where(acc > 0, acc, jnp.exp(jnp.minimum(acc, 0.0)) - 1.0)
    o_ref[...] = acc


def _mm(a, w, bias=None, act=None, bm=1000):
    M, K = a.shape
    F = w.shape[1]
    if bias is None:
        bias = jnp.zeros((F,), jnp.float32)
    grid = (M // bm,)
    return pl.pallas_call(
        functools.partial(_mm_body, act=act, scale_rows=None),
        grid=grid,
        in_specs=[
            pl.BlockSpec((bm, K), lambda i: (i, 0)),
            pl.BlockSpec((K, F), lambda i: (0, 0)),
            pl.BlockSpec((F,), lambda i: (0,)),
        ],
        out_specs=pl.BlockSpec((bm, F), lambda i: (i, 0)),
        out_shape=jax.ShapeDtypeStruct((M, F), jnp.float32),
    )(a, w, bias)


def _pairdot_body(a_ref, b_ref, o_ref):
    o_ref[...] = jnp.sum(a_ref[...] * b_ref[...], axis=1, keepdims=True)


def _pairdot(a, b, bp=2000):
    P, F = a.shape
    out = pl.pallas_call(
        _pairdot_body,
        grid=(P // bp,),
        in_specs=[
            pl.BlockSpec((bp, F), lambda i: (i, 0)),
            pl.BlockSpec((bp, F), lambda i: (i, 0)),
        ],
        out_specs=pl.BlockSpec((bp, 1), lambda i: (i, 0)),
        out_shape=jax.ShapeDtypeStruct((P, 1), jnp.float32),
    )(a, b)
    return out[:, 0]


def kernel(x, edge_index, hyperedge_index, pos_edges, neg_edges, W1, b1, W2, b2,
           Wh1, bh1, Wh2, bh2, gate, Wdec, Pw1, Pb1, Pw2, Pb2):
    N = x.shape[0]
    M = 2000
    elu = jax.nn.elu

    src = edge_index[0]
    dst = edge_index[1]
    deg = jnp.zeros((N,), jnp.float32).at[dst].add(1.0) + 1.0
    dinv = deg ** -0.5

    def gcn_agg(h):
        hn = h * dinv[:, None]
        agg = jnp.zeros_like(hn).at[dst].add(hn[src])
        return (agg + hn) * dinv[:, None]

    node = hyperedge_index[0]
    he = hyperedge_index[1]
    D = jnp.zeros((N,), jnp.float32).at[node].add(1.0)
    Dinv = jnp.where(D > 0, 1.0 / jnp.where(D > 0, D, 1.0), 0.0)
    B = jnp.zeros((M,), jnp.float32).at[he].add(1.0)
    Binv = jnp.where(B > 0, 1.0 / jnp.where(B > 0, B, 1.0), 0.0)

    def hyper_agg(h):
        eagg = jnp.zeros((M, h.shape[1]), h.dtype).at[he].add(h[node])
        eagg = eagg * Binv[:, None]
        out = jnp.zeros_like(h).at[node].add(eagg[he])
        return out * Dinv[:, None]

    h1 = _mm(x, W1)
    xs1 = elu(gcn_agg(h1) + b1)
    h2 = _mm(xs1, W2)
    x_s = gcn_agg(h2) + b2
    g1 = _mm(x, Wh1)
    xd1 = elu(hyper_agg(g1) + bh1)
    g2 = _mm(xd1, Wh2)
    x_d = hyper_agg(g2) + bh2

    alpha = jax.nn.sigmoid(gate)[0]
    z = alpha * x_s + (1.0 - alpha) * x_d

    zw = _mm(z, Wdec)
    pos_scores = _pairdot(zw[pos_edges[0]], z[pos_edges[1]])
    neg_scores = _pairdot(zw[neg_edges[0]], z[neg_edges[1]])

    proj_s = _mm(_mm(x_s, Pw1, Pb1, act="elu"), Pw2, Pb2)
    proj_d = _mm(_mm(x_d, Pw1, Pb1, act="elu"), Pw2, Pb2)
    return (pos_scores, neg_scores, proj_s, proj_d)
